# baseline (speedup 1.0000x reference)
import jax
import jax.numpy as jnp
from jax.experimental import pallas as pl
from jax.experimental.pallas import tpu as pltpu

N_EXP = 64
DIM = 1024
DGCN = 256
N_LOOP = 1024
N = N_EXP + 1

TILE_G = 128


def _kernel(ei_hbm, ew_hbm, X_ref, Wst_ref, p_ref, x_ref, Wm_ref, W0_ref,
            W1_ref, out_ref,
            dst_s, src_s, ew_s, Sx_s, Aaug_s, W1b_s, Wmb_s, W0b_s, P_s, sem):
    i = pl.program_id(0)
    e_pad = dst_s.shape[1]

    @pl.when(i == 0)
    def _build_tables():
        for hbm, dstref in ((ei_hbm.at[0:1, 0:e_pad], dst_s),
                            (ei_hbm.at[1:2, 0:e_pad], src_s),
                            (ew_hbm.at[pl.ds(0, e_pad)], ew_s)):
            cp = pltpu.make_async_copy(hbm, dstref, sem)
            cp.start()
            cp.wait()
        row_ids = jax.lax.broadcasted_iota(jnp.int32, (N, e_pad), 0)
        oh_dst_w = jnp.where(row_ids == dst_s[:], ew_s[:][None, :], 0.0)
        oh_src = (row_ids == src_s[:]).astype(jnp.float32)
        A = jax.lax.dot_general(oh_dst_w, oh_src, (((1,), (1,)), ((), ())),
                                preferred_element_type=jnp.float32)

        exp = jax.nn.relu(jnp.dot(X_ref[:], Wst_ref[:],
                                  preferred_element_type=jnp.float32))
        EW0 = jnp.dot(exp, W0_ref[:], preferred_element_type=jnp.float32)
        S = jnp.dot(A[:, :N_EXP], EW0, preferred_element_type=jnp.float32)
        scale = A[:, N_EXP:]
        Sx_s[:] = S / scale
        a_eh = scale[:N_EXP]
        a_hh = scale[N_EXP:]
        Aaug_s[:] = jnp.concatenate(
            [A[:N_EXP, :N_EXP] * scale[:N_EXP, 0][None, :], a_eh * a_hh],
            axis=1).astype(jnp.bfloat16)
        W1b_s[:] = W1_ref[:].astype(jnp.bfloat16)
        Wmb_s[:] = Wm_ref[:].astype(jnp.bfloat16)
        W0b_s[:] = W0_ref[:].astype(jnp.bfloat16)
        r_g = jax.lax.broadcasted_iota(jnp.int32, (TILE_G, DGCN, TILE_G), 0)
        c_g = jax.lax.broadcasted_iota(jnp.int32, (TILE_G, DGCN, TILE_G), 2)
        p3 = jnp.broadcast_to(p_ref[:][:, :, None], (TILE_G, DGCN, TILE_G))
        P_s[:] = jnp.where(r_g == c_g, p3, 0.0).astype(
            jnp.bfloat16).reshape(TILE_G * DGCN, TILE_G)

    g = x_ref.shape[0]
    xb = x_ref[:].astype(jnp.bfloat16)
    h = jax.nn.relu(jnp.dot(xb, Wmb_s[:],
                            preferred_element_type=jnp.float32))
    u = jnp.dot(h.astype(jnp.bfloat16), W0b_s[:],
                preferred_element_type=jnp.float32)

    r = jax.nn.relu(
        jnp.broadcast_to(Sx_s[:][:, None, :], (N, g, DGCN))
        + jnp.broadcast_to(u[None, :, :], (N, g, DGCN))
    ).astype(jnp.bfloat16).reshape(N * g, DGCN)

    t2 = jnp.dot(r, W1b_s[:], preferred_element_type=jnp.float32)
    t2b = t2.astype(jnp.bfloat16).reshape(N, g * DGCN)

    agg = jnp.dot(Aaug_s[:], t2b, preferred_element_type=jnp.float32)
    y2 = jax.nn.relu(agg).astype(jnp.bfloat16)

    out_ref[:] = jnp.dot(y2, P_s[:], preferred_element_type=jnp.float32)


@jax.jit
def kernel(x, X, W_mlp, W_struct, W_proj, W_gcn0, W_gcn1,
           edge_weight, edge_index):
    e_tot = edge_index.shape[1]
    e = e_tot // N_LOOP
    e_pad = min(e_tot, max(128, -(-e // 128) * 128))

    rep = lambda i: (0, 0)
    out = pl.pallas_call(
        _kernel,
        grid=(N_LOOP // TILE_G,),
        in_specs=[
            pl.BlockSpec(memory_space=pltpu.MemorySpace.HBM),
            pl.BlockSpec(memory_space=pltpu.MemorySpace.HBM),
            pl.BlockSpec((N_EXP, DIM), rep),
            pl.BlockSpec((DIM, DGCN), rep),
            pl.BlockSpec((1, DGCN), rep),
            pl.BlockSpec((TILE_G, DIM), lambda i: (i, 0)),
            pl.BlockSpec((DIM, DGCN), rep),
            pl.BlockSpec((DGCN, DGCN), rep),
            pl.BlockSpec((DGCN, DGCN), rep),
        ],
        out_specs=pl.BlockSpec((N_EXP, TILE_G), lambda i: (0, i)),
        out_shape=jax.ShapeDtypeStruct((N_EXP, N_LOOP), jnp.float32),
        scratch_shapes=[
            pltpu.VMEM((1, e_pad), jnp.int32),
            pltpu.VMEM((1, e_pad), jnp.int32),
            pltpu.VMEM((e_pad,), jnp.float32),
            pltpu.VMEM((N, DGCN), jnp.float32),
            pltpu.VMEM((N_EXP, N), jnp.bfloat16),
            pltpu.VMEM((DGCN, DGCN), jnp.bfloat16),
            pltpu.VMEM((DIM, DGCN), jnp.bfloat16),
            pltpu.VMEM((DGCN, DGCN), jnp.bfloat16),
            pltpu.VMEM((TILE_G * DGCN, TILE_G), jnp.bfloat16),
            pltpu.SemaphoreType.DMA,
        ],
    )(edge_index, edge_weight, X, W_struct, W_proj.reshape(1, DGCN), x,
      W_mlp, W_gcn0, W_gcn1)
    return out.T


# speedup vs baseline: 1.0087x; 1.0087x over previous
import jax
import jax.numpy as jnp
from jax.experimental import pallas as pl
from jax.experimental.pallas import tpu as pltpu

N_EXP = 64
DIM = 1024
DGCN = 256
N_LOOP = 1024
N = N_EXP + 1

TILE_G = 128


def _kernel(ei_hbm, ew_hbm, X_hbm, Wst_hbm, p_hbm, Wm_hbm, W0_hbm, W1_hbm,
            x_ref, out_ref,
            dst_s, src_s, ew_s, X_s, Wst_s, p_s, Wm_s, W0_s, W1_s,
            Sx_s, Aaug_s, W1b_s, Wmb_s, W0b_s, P_s, sem):
    i = pl.program_id(0)
    e_pad = dst_s.shape[1]

    @pl.when(i == 0)
    def _build_tables():
        copies = [
            pltpu.make_async_copy(ei_hbm.at[0:1, 0:e_pad], dst_s, sem),
            pltpu.make_async_copy(ei_hbm.at[1:2, 0:e_pad], src_s, sem),
            pltpu.make_async_copy(ew_hbm.at[pl.ds(0, e_pad)], ew_s, sem),
            pltpu.make_async_copy(X_hbm, X_s, sem),
            pltpu.make_async_copy(Wst_hbm, Wst_s, sem),
            pltpu.make_async_copy(p_hbm, p_s, sem),
            pltpu.make_async_copy(Wm_hbm, Wm_s, sem),
            pltpu.make_async_copy(W0_hbm, W0_s, sem),
            pltpu.make_async_copy(W1_hbm, W1_s, sem),
        ]
        for c in copies:
            c.start()
        for c in copies:
            c.wait()
        row_ids = jax.lax.broadcasted_iota(jnp.int32, (N, e_pad), 0)
        oh_dst_w = jnp.where(row_ids == dst_s[:], ew_s[:][None, :], 0.0)
        oh_src = (row_ids == src_s[:]).astype(jnp.float32)
        A = jax.lax.dot_general(oh_dst_w, oh_src, (((1,), (1,)), ((), ())),
                                preferred_element_type=jnp.float32)

        exp = jax.nn.relu(jnp.dot(X_s[:], Wst_s[:],
                                  preferred_element_type=jnp.float32))
        EW0 = jnp.dot(exp, W0_s[:], preferred_element_type=jnp.float32)
        S = jnp.dot(A[:, :N_EXP], EW0, preferred_element_type=jnp.float32)
        scale = A[:, N_EXP:]
        Sx_s[:] = S / scale
        a_eh = scale[:N_EXP]
        a_hh = scale[N_EXP:]
        Aaug_s[:] = jnp.concatenate(
            [A[:N_EXP, :N_EXP] * scale[:N_EXP, 0][None, :], a_eh * a_hh],
            axis=1).astype(jnp.bfloat16)
        W1b_s[:] = W1_s[:].astype(jnp.bfloat16)
        Wmb_s[:] = Wm_s[:].astype(jnp.bfloat16)
        W0b_s[:] = W0_s[:].astype(jnp.bfloat16)
        r_g = jax.lax.broadcasted_iota(jnp.int32, (TILE_G, DGCN, TILE_G), 0)
        c_g = jax.lax.broadcasted_iota(jnp.int32, (TILE_G, DGCN, TILE_G), 2)
        p3 = jnp.broadcast_to(p_s[:][:, :, None], (TILE_G, DGCN, TILE_G))
        P_s[:] = jnp.where(r_g == c_g, p3, 0.0).astype(
            jnp.bfloat16).reshape(TILE_G * DGCN, TILE_G)

    g = x_ref.shape[0]
    xb = x_ref[:].astype(jnp.bfloat16)
    h = jax.nn.relu(jnp.dot(xb, Wmb_s[:],
                            preferred_element_type=jnp.float32))
    u = jnp.dot(h.astype(jnp.bfloat16), W0b_s[:],
                preferred_element_type=jnp.float32)

    r = jax.nn.relu(
        jnp.broadcast_to(Sx_s[:][:, None, :], (N, g, DGCN))
        + jnp.broadcast_to(u[None, :, :], (N, g, DGCN))
    ).astype(jnp.bfloat16).reshape(N * g, DGCN)

    t2 = jnp.dot(r, W1b_s[:], preferred_element_type=jnp.float32)
    t2b = t2.astype(jnp.bfloat16).reshape(N, g * DGCN)

    agg = jnp.dot(Aaug_s[:], t2b, preferred_element_type=jnp.float32)
    y2 = jax.nn.relu(agg).astype(jnp.bfloat16)

    out_ref[:] = jnp.dot(y2, P_s[:], preferred_element_type=jnp.float32)


@jax.jit
def kernel(x, X, W_mlp, W_struct, W_proj, W_gcn0, W_gcn1,
           edge_weight, edge_index):
    e_tot = edge_index.shape[1]
    e = e_tot // N_LOOP
    e_pad = min(e_tot, max(128, -(-e // 128) * 128))

    hbm = pl.BlockSpec(memory_space=pltpu.MemorySpace.HBM)
    out = pl.pallas_call(
        _kernel,
        grid=(N_LOOP // TILE_G,),
        in_specs=[hbm] * 8 + [pl.BlockSpec((TILE_G, DIM), lambda i: (i, 0))],
        out_specs=pl.BlockSpec((N_EXP, TILE_G), lambda i: (0, i)),
        out_shape=jax.ShapeDtypeStruct((N_EXP, N_LOOP), jnp.float32),
        scratch_shapes=[
            pltpu.VMEM((1, e_pad), jnp.int32),
            pltpu.VMEM((1, e_pad), jnp.int32),
            pltpu.VMEM((e_pad,), jnp.float32),
            pltpu.VMEM((N_EXP, DIM), jnp.float32),
            pltpu.VMEM((DIM, DGCN), jnp.float32),
            pltpu.VMEM((1, DGCN), jnp.float32),
            pltpu.VMEM((DIM, DGCN), jnp.float32),
            pltpu.VMEM((DGCN, DGCN), jnp.float32),
            pltpu.VMEM((DGCN, DGCN), jnp.float32),
            pltpu.VMEM((N, DGCN), jnp.float32),
            pltpu.VMEM((N_EXP, N), jnp.bfloat16),
            pltpu.VMEM((DGCN, DGCN), jnp.bfloat16),
            pltpu.VMEM((DIM, DGCN), jnp.bfloat16),
            pltpu.VMEM((DGCN, DGCN), jnp.bfloat16),
            pltpu.VMEM((TILE_G * DGCN, TILE_G), jnp.bfloat16),
            pltpu.SemaphoreType.DMA,
        ],
    )(edge_index, edge_weight, X, W_struct, W_proj.reshape(1, DGCN),
      W_mlp, W_gcn0, W_gcn1, x)
    return out.T
